# baseline (speedup 1.0000x reference)
import functools

import jax
import jax.numpy as jnp
from jax import lax
from jax.experimental import pallas as pl
from jax.experimental.pallas import tpu as pltpu
from jax.experimental.pallas import tpu_sc as plsc

B = 512
K = 16384
D = 256
TOPK = 8
QBLK = 64
NB = B // QBLK


def _topk_tc_body(q_ref, keys_ref, idx_ref, ksq_ref, buf_ref):
    i = pl.program_id(0)

    @pl.when(i == 0)
    def _():
        kv = keys_ref[...]
        ksq_ref[...] = jnp.sum(kv * kv, axis=1)

    cross = lax.dot_general(
        -2.0 * q_ref[...], keys_ref[...],
        dimension_numbers=(((1,), (1,)), ((), ())),
        preferred_element_type=jnp.float32,
    )
    buf_ref[i % 2] = ksq_ref[...][None, :] + cross

    s = buf_ref[(i + 1) % 2][...]
    iota = lax.broadcasted_iota(jnp.int32, (QBLK, K), 1)
    cols = []
    for t in range(TOPK):
        it = jnp.argmin(s, axis=1)
        cols.append(it[:, None].astype(jnp.int32))
        if t + 1 < TOPK:
            s = jnp.where(iota == it[:, None].astype(jnp.int32), jnp.inf, s)
    idx_ref[...] = jnp.concatenate(cols, axis=1)


def _topk_indices(query, keys):
    return pl.pallas_call(
        _topk_tc_body,
        grid=(NB + 1,),
        in_specs=[
            pl.BlockSpec((QBLK, D), lambda i: (jnp.minimum(i, NB - 1), 0)),
            pl.BlockSpec((K, D), lambda i: (0, 0)),
        ],
        out_specs=pl.BlockSpec((QBLK, TOPK), lambda i: (i, 0)),
        out_shape=jax.ShapeDtypeStruct((B + QBLK, TOPK), jnp.int32),
        scratch_shapes=[
            pltpu.VMEM((K,), jnp.float32),
            pltpu.VMEM((2, QBLK, K), jnp.float32),
        ],
    )(query, keys)


def _make_sc_gather(n_rows):
    info = plsc.get_sparse_core_info()
    nw = info.num_cores * info.num_subcores
    rows_per_w = n_rows // nw
    mesh = plsc.VectorSubcoreMesh(core_axis_name="c", subcore_axis_name="s")

    @functools.partial(
        pl.kernel, mesh=mesh,
        out_type=jax.ShapeDtypeStruct((n_rows, D), jnp.float32),
        scratch_types=[
            pltpu.VMEM((rows_per_w,), jnp.int32),
            pltpu.VMEM((rows_per_w, D), jnp.float32),
            pltpu.SemaphoreType.DMA,
        ],
    )
    def gather_k(idx_hbm, table_hbm, out_hbm, idx_v, rows_v, sem):
        wid = lax.axis_index("s") * info.num_cores + lax.axis_index("c")
        base = wid * rows_per_w
        pltpu.sync_copy(idx_hbm.at[pl.ds(base, rows_per_w)], idx_v)
        pltpu.async_copy(table_hbm.at[idx_v], rows_v, sem).wait()
        pltpu.sync_copy(rows_v, out_hbm.at[pl.ds(base, rows_per_w)])

    return gather_k


def kernel(query, keys, vals, top_k):
    idx = _topk_indices(query, keys)[QBLK:]
    flat_idx = idx.reshape(B * TOPK)
    rows = _make_sc_gather(B * TOPK)(flat_idx, vals)
    return rows.reshape(B, TOPK, D)


# speedup vs baseline: 1.2658x; 1.2658x over previous
import functools

import jax
import jax.numpy as jnp
from jax import lax
from jax.experimental import pallas as pl
from jax.experimental.pallas import tpu as pltpu
from jax.experimental.pallas import tpu_sc as plsc

B = 512
K = 16384
D = 256
TOPK = 8
QBLK = 128
NB = B // QBLK


def _topk_tc_body(q_ref, keys_ref, idx_ref, ksq_ref):
    @pl.when(pl.program_id(0) == 0)
    def _():
        kv = keys_ref[...]
        ksq_ref[...] = jnp.sum(kv * kv, axis=1)

    cross = lax.dot_general(
        -2.0 * q_ref[...], keys_ref[...],
        dimension_numbers=(((1,), (1,)), ((), ())),
        preferred_element_type=jnp.float32,
    )
    s = ksq_ref[...][None, :] + cross
    iota = lax.broadcasted_iota(jnp.int32, (QBLK, K), 1)
    cols = []
    for t in range(TOPK):
        it = jnp.argmin(s, axis=1)
        cols.append(it[:, None].astype(jnp.int32))
        if t + 1 < TOPK:
            s = jnp.where(iota == it[:, None].astype(jnp.int32), jnp.inf, s)
    idx_ref[...] = jnp.concatenate(cols, axis=1)


def _topk_indices(query, keys):
    return pl.pallas_call(
        _topk_tc_body,
        grid=(NB,),
        in_specs=[
            pl.BlockSpec((QBLK, D), lambda i: (i, 0)),
            pl.BlockSpec((K, D), lambda i: (0, 0)),
        ],
        out_specs=pl.BlockSpec((QBLK, TOPK), lambda i: (i, 0)),
        out_shape=jax.ShapeDtypeStruct((B, TOPK), jnp.int32),
        scratch_shapes=[pltpu.VMEM((K,), jnp.float32)],
    )(query, keys)


def _make_sc_gather(n_rows):
    info = plsc.get_sparse_core_info()
    nw = info.num_cores * info.num_subcores
    rows_per_w = n_rows // nw
    mesh = plsc.VectorSubcoreMesh(core_axis_name="c", subcore_axis_name="s")

    @functools.partial(
        pl.kernel, mesh=mesh,
        out_type=jax.ShapeDtypeStruct((n_rows, D), jnp.float32),
        scratch_types=[
            pltpu.VMEM((rows_per_w,), jnp.int32),
            pltpu.VMEM((rows_per_w, D), jnp.float32),
            pltpu.SemaphoreType.DMA,
        ],
    )
    def gather_k(idx_hbm, table_hbm, out_hbm, idx_v, rows_v, sem):
        wid = lax.axis_index("s") * info.num_cores + lax.axis_index("c")
        base = wid * rows_per_w
        pltpu.sync_copy(idx_hbm.at[pl.ds(base, rows_per_w)], idx_v)
        pltpu.async_copy(table_hbm.at[idx_v], rows_v, sem).wait()
        pltpu.sync_copy(rows_v, out_hbm.at[pl.ds(base, rows_per_w)])

    return gather_k


def kernel(query, keys, vals, top_k):
    idx = _topk_indices(query, keys)
    flat_idx = idx.reshape(B * TOPK)
    rows = _make_sc_gather(B * TOPK)(flat_idx, vals)
    return rows.reshape(B, TOPK, D)
